# baseline (speedup 1.0000x reference)
import jax
import jax.numpy as jnp
from jax.experimental import pallas as pl
from jax.experimental.pallas import tpu as pltpu

_TN = 2048
_NBUF = 6
_N = 100000
_NFULL = _N // _TN


def _tile_copy(obuf, o_hbm, sem, slot, step):
    return pltpu.make_async_copy(
        obuf.at[slot],
        o_hbm.at[:, pl.ds(step * _TN, _TN)],
        sem.at[slot],
    )


def _dot(x, m):
    return jax.lax.dot_general(
        x,
        m,
        dimension_numbers=(((1,), (1,)), ((), ())),
        preferred_element_type=jnp.float32,
    )


def _main_kernel(x_ref, m_ref, o_hbm, obuf, sem):
    i = pl.program_id(0)
    slot = jax.lax.rem(i, _NBUF)

    @pl.when(i >= _NBUF)
    def _wait_prev():
        _tile_copy(obuf, o_hbm, sem, slot, i - _NBUF).wait()

    obuf[slot] = _dot(x_ref[...], m_ref[...])
    _tile_copy(obuf, o_hbm, sem, slot, i).start()

    @pl.when(i == _NFULL - 1)
    def _drain():
        for j in range(_NBUF):
            step = _NFULL - _NBUF + j
            _tile_copy(obuf, o_hbm, sem, step % _NBUF, step).wait()


def _tail_kernel(x_ref, m_ref, o_aliased, o_ref):
    del o_aliased
    o_ref[...] = _dot(x_ref[...], m_ref[...])


def kernel(inputs, targets, mem, epoch):
    del targets, epoch
    m, k = inputs.shape
    n = mem.shape[0]
    main = pl.pallas_call(
        _main_kernel,
        grid=(_NFULL,),
        in_specs=[
            pl.BlockSpec((m, k), lambda i: (0, 0)),
            pl.BlockSpec((_TN, k), lambda i: (i, 0)),
        ],
        out_specs=pl.BlockSpec(memory_space=pltpu.MemorySpace.HBM),
        out_shape=jax.ShapeDtypeStruct((m, n), jnp.float32),
        scratch_shapes=[
            pltpu.VMEM((_NBUF, m, _TN), jnp.float32),
            pltpu.SemaphoreType.DMA((_NBUF,)),
        ],
        compiler_params=pltpu.CompilerParams(
            dimension_semantics=("arbitrary",),
        ),
    )(inputs, mem)
    return pl.pallas_call(
        _tail_kernel,
        grid=(1,),
        in_specs=[
            pl.BlockSpec((m, k), lambda i: (0, 0)),
            pl.BlockSpec((_TN, k), lambda i: (_NFULL, 0)),
            pl.BlockSpec(memory_space=pltpu.MemorySpace.HBM),
        ],
        out_specs=pl.BlockSpec((m, _TN), lambda i: (0, _NFULL)),
        out_shape=jax.ShapeDtypeStruct((m, n), jnp.float32),
        input_output_aliases={2: 0},
    )(inputs, mem, main)


# speedup vs baseline: 1.0063x; 1.0063x over previous
import jax
import jax.numpy as jnp
from jax.experimental import pallas as pl
from jax.experimental.pallas import tpu as pltpu

_TN = 2048


def _logits_kernel(x_ref, m_ref, o_ref):
    o_ref[...] = jax.lax.dot_general(
        x_ref[...],
        m_ref[...],
        dimension_numbers=(((1,), (1,)), ((), ())),
        preferred_element_type=jnp.float32,
    )


def kernel(inputs, targets, mem, epoch):
    del targets, epoch
    m, k = inputs.shape
    n = mem.shape[0]
    return pl.pallas_call(
        _logits_kernel,
        grid=(pl.cdiv(n, _TN),),
        in_specs=[
            pl.BlockSpec((m, k), lambda i: (0, 0)),
            pl.BlockSpec((_TN, k), lambda i: (i, 0)),
        ],
        out_specs=pl.BlockSpec((m, _TN), lambda i: (0, i)),
        out_shape=jax.ShapeDtypeStruct((m, n), jnp.float32),
        compiler_params=pltpu.CompilerParams(
            dimension_semantics=("parallel",),
        ),
    )(inputs, mem)


# speedup vs baseline: 1.0181x; 1.0118x over previous
import jax
import jax.numpy as jnp
from jax.experimental import pallas as pl
from jax.experimental.pallas import tpu as pltpu

_TN = 2048
_NBUF = 6
_N = 100000
_NFULL = _N // _TN


def _tile_copy(obuf, o_hbm, sem, slot, step):
    return pltpu.make_async_copy(
        obuf.at[slot],
        o_hbm.at[:, pl.ds(step * _TN, _TN)],
        sem.at[slot],
    )


def _dot(x, m):
    return jax.lax.dot_general(
        x,
        m,
        dimension_numbers=(((1,), (1,)), ((), ())),
        preferred_element_type=jnp.float32,
    )


def _main_kernel(x_ref, m_ref, o_hbm, obuf, sem):
    i = pl.program_id(0)
    slot = jax.lax.rem(i, _NBUF)

    @pl.when(i >= _NBUF)
    def _wait_prev():
        _tile_copy(obuf, o_hbm, sem, slot, i - _NBUF).wait()

    obuf[slot] = _dot(x_ref[...], m_ref[...])
    for s in range(_NBUF):

        @pl.when(slot == s)
        def _start():
            _tile_copy(obuf, o_hbm, sem, s, i).start(priority=s % 2)

    @pl.when(i == _NFULL - 1)
    def _drain():
        for j in range(_NBUF):
            step = _NFULL - _NBUF + j
            _tile_copy(obuf, o_hbm, sem, step % _NBUF, step).wait()


def _tail_kernel(x_ref, m_ref, o_aliased, o_ref):
    del o_aliased
    o_ref[...] = _dot(x_ref[...], m_ref[...])


def kernel(inputs, targets, mem, epoch):
    del targets, epoch
    m, k = inputs.shape
    n = mem.shape[0]
    main = pl.pallas_call(
        _main_kernel,
        grid=(_NFULL,),
        in_specs=[
            pl.BlockSpec((m, k), lambda i: (0, 0)),
            pl.BlockSpec((_TN, k), lambda i: (i, 0)),
        ],
        out_specs=pl.BlockSpec(memory_space=pltpu.MemorySpace.HBM),
        out_shape=jax.ShapeDtypeStruct((m, n), jnp.float32),
        scratch_shapes=[
            pltpu.VMEM((_NBUF, m, _TN), jnp.float32),
            pltpu.SemaphoreType.DMA((_NBUF,)),
        ],
        compiler_params=pltpu.CompilerParams(
            dimension_semantics=("arbitrary",),
        ),
    )(inputs, mem)
    return pl.pallas_call(
        _tail_kernel,
        grid=(1,),
        in_specs=[
            pl.BlockSpec((m, k), lambda i: (0, 0)),
            pl.BlockSpec((_TN, k), lambda i: (_NFULL, 0)),
            pl.BlockSpec(memory_space=pltpu.MemorySpace.HBM),
        ],
        out_specs=pl.BlockSpec((m, _TN), lambda i: (0, _NFULL)),
        out_shape=jax.ShapeDtypeStruct((m, n), jnp.float32),
        input_output_aliases={2: 0},
    )(inputs, mem, main)
